# baseline (speedup 1.0000x reference)
import functools

import jax
import jax.numpy as jnp
from jax import lax
from jax.experimental import pallas as pl
from jax.experimental.pallas import tpu as pltpu
from jax.experimental.pallas import tpu_sc as plsc

EMB = 32
NC = 2
NS = 16
NW = NC * NS
IDX_CHUNK = 128

_DNUMS = lax.GatherDimensionNumbers(
    offset_dims=(), collapsed_slice_dims=(0,), start_index_map=(0,))


def _rot(v, idx):
    return lax.gather(v, idx[:, None], _DNUMS, slice_sizes=(1,),
                      mode=lax.GatherScatterMode.PROMISE_IN_BOUNDS)


def kernel(u_id, i_id, user_table, item_table):
    B = u_id.shape[0]
    b_per_w = B // NW
    chunks = b_per_w // IDX_CHUNK
    u2 = u_id.reshape(NW * chunks, IDX_CHUNK).astype(jnp.int32)
    i2 = i_id.reshape(NW * chunks, IDX_CHUNK).astype(jnp.int32)
    mesh = plsc.VectorSubcoreMesh(core_axis_name="c", subcore_axis_name="s")

    @functools.partial(
        pl.kernel,
        out_type=jax.ShapeDtypeStruct((B,), jnp.float32),
        mesh=mesh,
        scratch_types=[
            pltpu.VMEM((chunks, IDX_CHUNK), jnp.int32),
            pltpu.VMEM((chunks, IDX_CHUNK), jnp.int32),
            pltpu.VMEM((b_per_w, EMB), jnp.float32),
            pltpu.VMEM((b_per_w, EMB), jnp.float32),
            pltpu.VMEM((b_per_w,), jnp.float32),
            pltpu.SemaphoreType.DMA,
        ],
        compiler_params=pltpu.CompilerParams(use_tc_tiling_on_sc=False),
    )
    def run(u2_hbm, i2_hbm, ut_hbm, it_hbm, out_hbm,
            uidx, iidx, urows, irows, outv, sem):
        wid = lax.axis_index("s") * NC + lax.axis_index("c")
        base = wid * b_per_w
        pltpu.sync_copy(u2_hbm.at[pl.ds(wid * chunks, chunks)], uidx)
        pltpu.sync_copy(i2_hbm.at[pl.ds(wid * chunks, chunks)], iidx)
        cps = []
        for c in range(chunks):
            dst = pl.ds(c * IDX_CHUNK, IDX_CHUNK)
            cps.append(pltpu.async_copy(ut_hbm.at[uidx.at[c]], urows.at[dst], sem))
            cps.append(pltpu.async_copy(it_hbm.at[iidx.at[c]], irows.at[dst], sem))
        for cp in cps:
            cp.wait()

        lane = lax.broadcasted_iota(jnp.int32, (16,), 0)
        idx8 = (lane + 8) & 15
        idx4 = (lane & 8) | ((lane + 4) & 7)
        idx2 = (lane & 12) | ((lane + 2) & 3)
        idx1 = lane ^ 1
        idxbr = (((lane & 1) << 3) | ((lane & 2) << 1)
                 | ((lane & 4) >> 1) | ((lane & 8) >> 3))
        stages = ((idx8, lane < 8), (idx4, (lane & 7) < 4),
                  (idx2, (lane & 3) < 2), (idx1, (lane & 1) == 0))

        def body(g, carry):
            vs = []
            for j in range(16):
                b = g * 16 + j
                vs.append(urows[b, pl.ds(0, 16)] * irows[b, pl.ds(0, 16)]
                          + urows[b, pl.ds(16, 16)] * irows[b, pl.ds(16, 16)])
            for idx, m in stages:
                vs = [jnp.where(m, vs[2 * j] + _rot(vs[2 * j], idx),
                                vs[2 * j + 1] + _rot(vs[2 * j + 1], idx))
                      for j in range(len(vs) // 2)]
            outv[pl.ds(g * 16, 16)] = _rot(vs[0], idxbr)
            return carry

        lax.fori_loop(0, b_per_w // 16, body, 0)
        pltpu.sync_copy(outv, out_hbm.at[pl.ds(base, b_per_w)])

    return run(u2, i2, user_table, item_table)
